# baseline (speedup 1.0000x reference)
import functools

import jax
import jax.numpy as jnp
from jax import lax
from jax.experimental import pallas as pl
from jax.experimental.pallas import tpu as pltpu
from jax.experimental.pallas import tpu_sc as plsc

R, D, N = 262144, 64, 16384
DP = 128
NC, NS, L = 2, 16, 16
W = NC * NS
RPT = R // W
SUB = 256
NSUB = RPT // SUB
IBLK = 4096
CAP = N + L

_mesh = plsc.VectorSubcoreMesh(
    core_axis_name="c", subcore_axis_name="s", num_cores=NC, num_subcores=NS
)


@functools.partial(
    pl.kernel,
    out_type=jax.ShapeDtypeStruct((R, D), jnp.float32),
    mesh=_mesh,
    scratch_types=[
        pltpu.VMEM((IBLK,), jnp.int32),
        pltpu.VMEM((CAP,), jnp.int32),
        pltpu.VMEM((CAP,), jnp.int32),
        pltpu.VMEM((SUB, D), jnp.float32),
        pltpu.VMEM((L, DP), jnp.float32),
        pltpu.VMEM((2 * L,), jnp.int32),
        pltpu.SemaphoreType.DMA,
    ],
    compiler_params=pltpu.CompilerParams(needs_layout_passes=False),
)
def _scatter_add(idx_hbm, a_hbm, bp_hbm, out_hbm,
                 idx_v, buf1, buf2, subslab, stage, rowbuf, sem):
    c = lax.axis_index("c")
    s = lax.axis_index("s")
    w = s * NC + c
    tile_base = w * RPT
    iota = lax.iota(jnp.int32, L)

    def blk(b, cnt):
        pltpu.sync_copy(idx_hbm.at[pl.ds(b * IBLK, IBLK)], idx_v)

        def f1(v, cnt):
            iv = idx_v[pl.ds(v * L, L)]
            li = iv - tile_base
            m = (li >= 0) & (li < RPT)
            packed = li * N + (b * IBLK + v * L + iota)
            plsc.store_compressed(buf1.at[pl.ds(cnt, L)], packed, mask=m)
            return cnt + plsc.all_reduce_population_count(m)[0]

        return lax.fori_loop(0, IBLK // L, f1, cnt)

    cnt1 = lax.fori_loop(0, N // IBLK, blk, jnp.int32(0))
    nv1 = (cnt1 + L - 1) // L

    def sub_body(j, carry):
        sub0 = j * SUB
        lo = sub0 * N
        hi = (sub0 + SUB) * N
        pltpu.sync_copy(a_hbm.at[pl.ds(tile_base + sub0, SUB)], subslab)

        def f2(v, cnt):
            pv = buf1[pl.ds(v * L, L)]
            m = (pv >= lo) & (pv < hi) & (v * L + iota < cnt1)
            plsc.store_compressed(buf2.at[pl.ds(cnt, L)], pv, mask=m)
            return cnt + plsc.all_reduce_population_count(m)[0]

        cnt2 = lax.fori_loop(0, nv1, f2, jnp.int32(0))
        buf2[pl.ds(cnt2, L)] = jnp.full((L,), -1, jnp.int32)

        def g(k, carry2):
            pv = buf2[pl.ds(k * L, L)]
            bpos = jnp.where(pv < 0, -1, pv & (N - 1))
            gidx = plsc.Indices(bpos, ignored_value=-1)
            pltpu.async_copy(bp_hbm.at[gidx], stage, sem).wait()
            lrow = (pv >> 14) - sub0
            rowbuf[pl.ds(0, L)] = lrow
            valid = jnp.minimum(cnt2 - k * L, L)

            def rbody(t, carry3):
                r = rowbuf[pl.ds(t, L)][0]
                for q in range(D // L):
                    acc = subslab[r, pl.ds(q * L, L)]
                    subslab[r, pl.ds(q * L, L)] = acc + stage[t, pl.ds(q * L, L)]
                return carry3

            lax.fori_loop(0, valid, rbody, jnp.int32(0))
            return carry2

        ng = (cnt2 + L - 1) // L
        lax.fori_loop(0, ng, g, jnp.int32(0))

        pltpu.sync_copy(subslab, out_hbm.at[pl.ds(tile_base + sub0, SUB)])
        return carry

    lax.fori_loop(0, NSUB, sub_body, jnp.int32(0))


def kernel(index, A, B):
    b_pad = jnp.pad(B, ((0, 0), (0, DP - D)))
    return _scatter_add(index.astype(jnp.int32), A, b_pad)


# speedup vs baseline: 1.0005x; 1.0005x over previous
import functools

import jax
import jax.numpy as jnp
from jax import lax
from jax.experimental import pallas as pl
from jax.experimental.pallas import tpu as pltpu
from jax.experimental.pallas import tpu_sc as plsc

R, D, N = 262144, 64, 16384
DP = 128
NC, NS, L = 2, 16, 16
W = NC * NS
RPT = R // W
SUB = 256
NSUB = RPT // SUB
IBLK = 4096
CAP = N + L

_mesh = plsc.VectorSubcoreMesh(
    core_axis_name="c", subcore_axis_name="s", num_cores=NC, num_subcores=NS
)


@functools.partial(
    pl.kernel,
    out_type=jax.ShapeDtypeStruct((R, D), jnp.float32),
    mesh=_mesh,
    scratch_types=[
        pltpu.VMEM((IBLK,), jnp.int32),
        pltpu.VMEM((CAP,), jnp.int32),
        pltpu.VMEM((CAP,), jnp.int32),
        pltpu.VMEM((SUB, D), jnp.float32),
        pltpu.VMEM((L, DP), jnp.float32),
        pltpu.VMEM((2 * L,), jnp.int32),
        pltpu.SemaphoreType.DMA,
    ],
    compiler_params=pltpu.CompilerParams(
        needs_layout_passes=False, skip_device_barrier=True
    ),
)
def _scatter_add(idx_hbm, a_hbm, bp_hbm, out_hbm,
                 idx_v, buf1, buf2, subslab, stage, rowbuf, sem):
    c = lax.axis_index("c")
    s = lax.axis_index("s")
    w = s * NC + c
    tile_base = w * RPT
    iota = lax.iota(jnp.int32, L)

    def blk(b, cnt):
        pltpu.sync_copy(idx_hbm.at[pl.ds(b * IBLK, IBLK)], idx_v)

        def f1(v, cnt):
            iv = idx_v[pl.ds(v * L, L)]
            li = iv - tile_base
            m = (li >= 0) & (li < RPT)
            packed = li * N + (b * IBLK + v * L + iota)
            plsc.store_compressed(buf1.at[pl.ds(cnt, L)], packed, mask=m)
            return cnt + plsc.all_reduce_population_count(m)[0]

        return lax.fori_loop(0, IBLK // L, f1, cnt)

    cnt1 = lax.fori_loop(0, N // IBLK, blk, jnp.int32(0))
    nv1 = (cnt1 + L - 1) // L

    def sub_body(j, carry):
        sub0 = j * SUB
        lo = sub0 * N
        hi = (sub0 + SUB) * N
        pltpu.sync_copy(a_hbm.at[pl.ds(tile_base + sub0, SUB)], subslab)

        def f2(v, cnt):
            pv = buf1[pl.ds(v * L, L)]
            m = (pv >= lo) & (pv < hi) & (v * L + iota < cnt1)
            plsc.store_compressed(buf2.at[pl.ds(cnt, L)], pv, mask=m)
            return cnt + plsc.all_reduce_population_count(m)[0]

        cnt2 = lax.fori_loop(0, nv1, f2, jnp.int32(0))
        buf2[pl.ds(cnt2, L)] = jnp.full((L,), -1, jnp.int32)

        def g(k, carry2):
            pv = buf2[pl.ds(k * L, L)]
            bpos = jnp.where(pv < 0, -1, pv & (N - 1))
            gidx = plsc.Indices(bpos, ignored_value=-1)
            pltpu.async_copy(bp_hbm.at[gidx], stage, sem).wait()
            lrow = (pv >> 14) - sub0
            rowbuf[pl.ds(0, L)] = lrow
            valid = jnp.minimum(cnt2 - k * L, L)

            def rbody(t, carry3):
                r = rowbuf[pl.ds(t, L)][0]
                for q in range(D // L):
                    acc = subslab[r, pl.ds(q * L, L)]
                    subslab[r, pl.ds(q * L, L)] = acc + stage[t, pl.ds(q * L, L)]
                return carry3

            lax.fori_loop(0, valid, rbody, jnp.int32(0))
            return carry2

        ng = (cnt2 + L - 1) // L
        lax.fori_loop(0, ng, g, jnp.int32(0))

        pltpu.sync_copy(subslab, out_hbm.at[pl.ds(tile_base + sub0, SUB)])
        return carry

    lax.fori_loop(0, NSUB, sub_body, jnp.int32(0))


def kernel(index, A, B):
    b_pad = jnp.pad(B, ((0, 0), (0, DP - D)))
    return _scatter_add(index.astype(jnp.int32), A, b_pad)


# speedup vs baseline: 2.1360x; 2.1349x over previous
import functools

import jax
import jax.numpy as jnp
from jax import lax
from jax.experimental import pallas as pl
from jax.experimental.pallas import tpu as pltpu
from jax.experimental.pallas import tpu_sc as plsc

R, D, N = 262144, 64, 16384
DP = 128
NC, NS, L = 2, 16, 16
W = NC * NS
CPT = R // W
CW = 256
NSLAB = CPT // CW
IBLK = 4096
CAP = N + L

_mesh = plsc.VectorSubcoreMesh(
    core_axis_name="c", subcore_axis_name="s", num_cores=NC, num_subcores=NS
)


@functools.partial(
    pl.kernel,
    out_type=jax.ShapeDtypeStruct((D, R), jnp.float32),
    mesh=_mesh,
    scratch_types=[
        pltpu.VMEM((IBLK,), jnp.int32),
        pltpu.VMEM((CAP,), jnp.int32),
        pltpu.VMEM((CAP,), jnp.int32),
        pltpu.VMEM((D, CW), jnp.float32),
        pltpu.VMEM((L, DP), jnp.float32),
        pltpu.VMEM((2 * L,), jnp.int32),
        pltpu.SemaphoreType.DMA,
    ],
    compiler_params=pltpu.CompilerParams(needs_layout_passes=False),
)
def _scatter_add(idx_hbm, at_hbm, bp_hbm, out_hbm,
                 idx_v, buf1, buf2, subslab, stage, rowbuf, sem):
    c = lax.axis_index("c")
    s = lax.axis_index("s")
    w = s * NC + c
    tile_base = w * CPT
    iota = lax.iota(jnp.int32, L)

    def blk(b, cnt):
        pltpu.sync_copy(idx_hbm.at[pl.ds(b * IBLK, IBLK)], idx_v)

        def f1(v, cnt):
            iv = idx_v[pl.ds(v * L, L)]
            li = iv - tile_base
            m = (li >= 0) & (li < CPT)
            packed = li * N + (b * IBLK + v * L + iota)
            plsc.store_compressed(buf1.at[pl.ds(cnt, L)], packed, mask=m)
            return cnt + plsc.all_reduce_population_count(m)[0]

        return lax.fori_loop(0, IBLK // L, f1, cnt)

    cnt1 = lax.fori_loop(0, N // IBLK, blk, jnp.int32(0))
    nv1 = (cnt1 + L - 1) // L

    def slab_body(j, carry):
        col0 = j * CW
        lo = col0 * N
        hi = (col0 + CW) * N
        pltpu.sync_copy(at_hbm.at[:, pl.ds(tile_base + col0, CW)], subslab)

        def f2(v, cnt):
            pv = buf1[pl.ds(v * L, L)]
            m = (pv >= lo) & (pv < hi) & (v * L + iota < cnt1)
            plsc.store_compressed(buf2.at[pl.ds(cnt, L)], pv, mask=m)
            return cnt + plsc.all_reduce_population_count(m)[0]

        cnt2 = lax.fori_loop(0, nv1, f2, jnp.int32(0))
        buf2[pl.ds(cnt2, L)] = jnp.full((L,), -1, jnp.int32)

        def g(k, carry2):
            pv = buf2[pl.ds(k * L, L)]
            bpos = jnp.where(pv < 0, -1, pv & (N - 1))
            gidx = plsc.Indices(bpos, ignored_value=-1)
            pltpu.async_copy(bp_hbm.at[gidx], stage, sem).wait()
            lcol = (pv >> 14) - col0
            rowbuf[pl.ds(0, L)] = lcol
            valid = jnp.minimum(cnt2 - k * L, L)

            def ubody(t, carry3):
                rc = rowbuf[pl.ds(t, L)][0]
                cols = jnp.full((L,), 0, jnp.int32) + rc
                for q in range(D // L):
                    plsc.addupdate_scatter(
                        subslab, [q * L + iota, cols],
                        stage[t, pl.ds(q * L, L)],
                    )
                return carry3

            lax.fori_loop(0, valid, ubody, jnp.int32(0))
            return carry2

        ng = (cnt2 + L - 1) // L
        lax.fori_loop(0, ng, g, jnp.int32(0))

        pltpu.sync_copy(subslab, out_hbm.at[:, pl.ds(tile_base + col0, CW)])
        return carry

    lax.fori_loop(0, NSLAB, slab_body, jnp.int32(0))


def kernel(index, A, B):
    b_pad = jnp.pad(B, ((0, 0), (0, DP - D)))
    out_t = _scatter_add(index.astype(jnp.int32), A.T, b_pad)
    return out_t.T


# speedup vs baseline: 2.6272x; 1.2299x over previous
import functools

import jax
import jax.numpy as jnp
from jax import lax
from jax.experimental import pallas as pl
from jax.experimental.pallas import tpu as pltpu
from jax.experimental.pallas import tpu_sc as plsc

R, D, N = 262144, 64, 16384
DP = 128
NC, NS, L = 2, 16, 16
W = NC * NS
CPT = R // W
CW = 256
NSLAB = CPT // CW
IBLK = 4096
CAP = N + L

_mesh = plsc.VectorSubcoreMesh(
    core_axis_name="c", subcore_axis_name="s", num_cores=NC, num_subcores=NS
)


@functools.partial(
    pl.kernel,
    out_type=jax.ShapeDtypeStruct((D, R), jnp.float32),
    mesh=_mesh,
    scratch_types=[
        pltpu.VMEM((IBLK,), jnp.int32),
        pltpu.VMEM((CAP,), jnp.int32),
        pltpu.VMEM((CAP,), jnp.int32),
        pltpu.VMEM((2, D, CW), jnp.float32),
        pltpu.VMEM((L, DP), jnp.float32),
        pltpu.VMEM((2 * L,), jnp.int32),
        pltpu.SemaphoreType.DMA,
        pltpu.SemaphoreType.DMA,
        pltpu.SemaphoreType.DMA,
        pltpu.SemaphoreType.DMA,
        pltpu.SemaphoreType.DMA,
    ],
    compiler_params=pltpu.CompilerParams(needs_layout_passes=False),
)
def _scatter_add(idx_hbm, at_hbm, bp_hbm, out_hbm,
                 idx_v, buf1, buf2, slabs, stage, rowbuf,
                 gsem, isem0, isem1, osem0, osem1):
    c = lax.axis_index("c")
    s = lax.axis_index("s")
    w = s * NC + c
    tile_base = w * CPT
    iota = lax.iota(jnp.int32, L)
    isem = (isem0, isem1)
    osem = (osem0, osem1)

    def blk(b, cnt):
        pltpu.sync_copy(idx_hbm.at[pl.ds(b * IBLK, IBLK)], idx_v)

        def f1(v, cnt):
            for u in range(2):
                iv = idx_v[pl.ds((2 * v + u) * L, L)]
                li = iv - tile_base
                m = (li >= 0) & (li < CPT)
                packed = li * N + (b * IBLK + (2 * v + u) * L + iota)
                plsc.store_compressed(buf1.at[pl.ds(cnt, L)], packed, mask=m)
                cnt = cnt + plsc.all_reduce_population_count(m)[0]
            return cnt

        return lax.fori_loop(0, IBLK // L // 2, f1, cnt)

    cnt1 = lax.fori_loop(0, N // IBLK, blk, jnp.int32(0))
    nv1 = (cnt1 + L - 1) // L

    def compute_slab(j, p):
        col0 = j * CW
        lo = col0 * N
        hi = (col0 + CW) * N
        sub = slabs.at[p]

        def f2(v, cnt):
            pv = buf1[pl.ds(v * L, L)]
            m = (pv >= lo) & (pv < hi) & (v * L + iota < cnt1)
            plsc.store_compressed(buf2.at[pl.ds(cnt, L)], pv, mask=m)
            return cnt + plsc.all_reduce_population_count(m)[0]

        cnt2 = lax.fori_loop(0, nv1, f2, jnp.int32(0))
        buf2[pl.ds(cnt2, L)] = jnp.full((L,), -1, jnp.int32)

        def g(k, carry2):
            pv = buf2[pl.ds(k * L, L)]
            bpos = jnp.where(pv < 0, -1, pv & (N - 1))
            gidx = plsc.Indices(bpos, ignored_value=-1)
            pltpu.async_copy(bp_hbm.at[gidx], stage, gsem).wait()
            lcol = (pv >> 14) - col0
            rowbuf[pl.ds(0, L)] = lcol
            valid = jnp.minimum(cnt2 - k * L, L)

            def ubody(t, carry3):
                rc = rowbuf[pl.ds(t, L)][0]
                cols = jnp.full((L,), 0, jnp.int32) + rc
                for q in range(D // L):
                    plsc.addupdate_scatter(
                        sub, [q * L + iota, cols],
                        stage[t, pl.ds(q * L, L)],
                    )
                return carry3

            lax.fori_loop(0, valid, ubody, jnp.int32(0))
            return carry2

        ng = (cnt2 + L - 1) // L
        lax.fori_loop(0, ng, g, jnp.int32(0))

    def issue_in(j, p):
        return pltpu.async_copy(
            at_hbm.at[:, pl.ds(tile_base + j * CW, CW)], slabs.at[p], isem[p]
        )

    def issue_out(j, p):
        return pltpu.async_copy(
            slabs.at[p], out_hbm.at[:, pl.ds(tile_base + j * CW, CW)], osem[p]
        )

    d_in = [None, None]
    d_out = [None, None]
    d_in[0] = issue_in(0, 0)
    for j in range(NSLAB):
        p = j & 1
        if j + 1 < NSLAB:
            if d_out[1 - p] is not None:
                d_out[1 - p].wait()
            d_in[1 - p] = issue_in(j + 1, 1 - p)
        d_in[p].wait()
        compute_slab(j, p)
        d_out[p] = issue_out(j, p)
    d_out[0].wait()
    d_out[1].wait()


def kernel(index, A, B):
    b_pad = jnp.pad(B, ((0, 0), (0, DP - D)))
    out_t = _scatter_add(index.astype(jnp.int32), A.T, b_pad)
    return out_t.T
